# baseline (speedup 1.0000x reference)
import jax
import jax.numpy as jnp
from jax.experimental import pallas as pl
from jax.experimental.pallas import tpu as pltpu

_NSEM = 8


def kernel(x, row_emb, col_emb):
    b = x.shape[0]
    h, w = x.shape[-2], x.shape[-1]
    d = row_emb.shape[1]

    def body(col_ref, row_ref, out_ref, scratch, sems):
        col = col_ref[:w, :]
        row = row_ref[:h, :]
        scratch[:, :, 0:d] = jnp.broadcast_to(col[None, :, :], (h, w, d))
        scratch[:, :, d:2 * d] = jnp.broadcast_to(row[:, None, :], (h, w, d))
        copies = [
            pltpu.make_async_copy(scratch, out_ref.at[i], sems.at[i % _NSEM])
            for i in range(b)
        ]
        for c in copies:
            c.start()
        for c in copies:
            c.wait()

    out = pl.pallas_call(
        body,
        out_specs=pl.BlockSpec(memory_space=pl.ANY),
        out_shape=jax.ShapeDtypeStruct((b, h, w, 2 * d), jnp.float32),
        scratch_shapes=[
            pltpu.VMEM((h, w, 2 * d), jnp.float32),
            pltpu.SemaphoreType.DMA((_NSEM,)),
        ],
    )(col_emb, row_emb)
    return jnp.transpose(out, (0, 3, 1, 2))


# speedup vs baseline: 1.0133x; 1.0133x over previous
import jax
import jax.numpy as jnp
from jax.experimental import pallas as pl

_BPG = 2


def kernel(x, row_emb, col_emb):
    b = x.shape[0]
    h, w = x.shape[-2], x.shape[-1]
    d = row_emb.shape[1]

    def body(col_ref, row_ref, out_ref):
        col = col_ref[:w, :]
        row = row_ref[:h, :]
        out_ref[:, :, :, 0:d] = jnp.broadcast_to(
            col[None, None, :, :], (_BPG, h, w, d))
        out_ref[:, :, :, d:2 * d] = jnp.broadcast_to(
            row[None, :, None, :], (_BPG, h, w, d))

    out = pl.pallas_call(
        body,
        grid=(b // _BPG,),
        in_specs=[
            pl.BlockSpec(col_emb.shape, lambda i: (0, 0)),
            pl.BlockSpec(row_emb.shape, lambda i: (0, 0)),
        ],
        out_specs=pl.BlockSpec((_BPG, h, w, 2 * d), lambda i: (i, 0, 0, 0)),
        out_shape=jax.ShapeDtypeStruct((b, h, w, 2 * d), jnp.float32),
    )(col_emb, row_emb)
    return jnp.transpose(out, (0, 3, 1, 2))
